# baseline (speedup 1.0000x reference)
import functools

import jax
import jax.numpy as jnp
from jax import lax
from jax.experimental import pallas as pl
from jax.experimental.pallas import tpu as pltpu
from jax.experimental.pallas import tpu_sc as plsc

_LANES = 16
_SLICE = 128
_NSL = 2
_CHUNK = _SLICE * _NSL


@functools.partial(jax.jit, static_argnames=("vocab",))
def _gather_concat(idx_flat, ori_w, add_w, *, vocab):
  n = idx_flat.shape[0]
  d = ori_w.shape[1]
  add_n = add_w.shape[0]
  info = plsc.get_sparse_core_info()
  nc, ns = info.num_cores, info.num_subcores
  nw = nc * ns
  assert n % (nw * _CHUNK * 2) == 0
  per_w = n // nw
  n_pairs = per_w // (_CHUNK * 2)

  mesh = plsc.VectorSubcoreMesh(core_axis_name="c", subcore_axis_name="s")

  buf_kinds = [
      pltpu.VMEM((_CHUNK,), jnp.int32),
      pltpu.VMEM((_NSL, _SLICE), jnp.int32),
      pltpu.VMEM((_NSL, _SLICE), jnp.int32),
      pltpu.VMEM((_CHUNK, d), jnp.float32),
      pltpu.VMEM((_CHUNK, d), jnp.float32),
      pltpu.SemaphoreType.DMA,
      pltpu.SemaphoreType.DMA,
  ]

  @functools.partial(
      pl.kernel,
      mesh=mesh,
      compiler_params=pltpu.CompilerParams(
          use_tc_tiling_on_sc=False, needs_layout_passes=False),
      out_type=jax.ShapeDtypeStruct((n, d), jnp.float32),
      scratch_types=buf_kinds + buf_kinds + [
          pltpu.SemaphoreType.DMA,
          pltpu.SemaphoreType.DMA,
      ],
  )
  def k(idx_hbm, ori_hbm, add_hbm, out_hbm,
        idxv_a, iori_a, iadd_a, obuf_a, abuf_a, semi_a, semo_a,
        idxv_b, iori_b, iadd_b, obuf_b, abuf_b, semi_b, semo_b,
        sem1, sem2):
    wid = lax.axis_index("s") * nc + lax.axis_index("c")
    base = wid * per_w
    lanes = lax.iota(jnp.int32, _LANES)

    def prep(idxv, iori, iadd):
      for i in range(_CHUNK // _LANES):
        v = idxv[pl.ds(i * _LANES, _LANES)]
        j = i // (_SLICE // _LANES)
        o = (i % (_SLICE // _LANES)) * _LANES
        iori[j, pl.ds(o, _LANES)] = jnp.minimum(v, vocab - 1)
        iadd[j, pl.ds(o, _LANES)] = jnp.where(
            v >= vocab, jnp.minimum(v - vocab, add_n - 1), lanes + i * _LANES)

    def gathers(iori, iadd, obuf, abuf):
      copies = []
      for j in range(_NSL):
        dst = pl.ds(j * _SLICE, _SLICE)
        copies.append(
            pltpu.async_copy(ori_hbm.at[iori.at[j]], obuf.at[dst], sem1))
        copies.append(
            pltpu.async_copy(add_hbm.at[iadd.at[j]], abuf.at[dst], sem2))
      return copies

    def merge(idxv, obuf, abuf):
      def body(i, c):
        v = idxv[pl.ds(i * _LANES, _LANES)]
        m = v >= vocab

        @pl.when(jnp.max(v) >= vocab)
        def _():
          rowv = lanes + i * _LANES
          for w in range(d):
            colv = jnp.full((_LANES,), w, dtype=jnp.int32)
            x = plsc.load_gather(abuf, [rowv, colv], mask=m)
            plsc.store_scatter(obuf, [rowv, colv], x, mask=m)

        return c

      lax.fori_loop(0, _CHUNK // _LANES, body, 0)

    def half(g2, g, idxv, iori, iadd, obuf, abuf, semi, semo,
             nxt_rb, nxt_idxv, nxt_semi, prefetch_guard):
      rb = base + g * _CHUNK
      pltpu.make_async_copy(idx_hbm.at[pl.ds(rb, _CHUNK)], idxv, semi).wait()
      prep(idxv, iori, iadd)
      @pl.when(g2 > 0)
      def _():
        pltpu.make_async_copy(obuf, out_hbm.at[pl.ds(rb, _CHUNK)],
                              semo).wait()

      copies = gathers(iori, iadd, obuf, abuf)
      @pl.when(prefetch_guard)
      def _():
        pltpu.async_copy(idx_hbm.at[pl.ds(nxt_rb, _CHUNK)], nxt_idxv,
                         nxt_semi)

      for c in copies:
        c.wait()
      merge(idxv, obuf, abuf)
      pltpu.async_copy(obuf, out_hbm.at[pl.ds(rb, _CHUNK)], semo)

    pltpu.async_copy(idx_hbm.at[pl.ds(base, _CHUNK)], idxv_a, semi_a)

    def pair_body(g2, carry):
      ga = 2 * g2
      gb = ga + 1
      half(g2, ga, idxv_a, iori_a, iadd_a, obuf_a, abuf_a, semi_a, semo_a,
           base + gb * _CHUNK, idxv_b, semi_b, gb < 2 * n_pairs)
      half(g2, gb, idxv_b, iori_b, iadd_b, obuf_b, abuf_b, semi_b, semo_b,
           base + (gb + 1) * _CHUNK, idxv_a, semi_a, g2 < n_pairs - 1)
      return carry

    lax.fori_loop(0, n_pairs, pair_body, 0)

    last_a = base + (2 * n_pairs - 2) * _CHUNK
    last_b = base + (2 * n_pairs - 1) * _CHUNK
    pltpu.make_async_copy(obuf_a, out_hbm.at[pl.ds(last_a, _CHUNK)],
                          semo_a).wait()
    pltpu.make_async_copy(obuf_b, out_hbm.at[pl.ds(last_b, _CHUNK)],
                          semo_b).wait()

  return k(idx_flat, ori_w, add_w)


def kernel(input, ori_w, add_w):
  b, l = input.shape
  vocab, d = ori_w.shape
  out = _gather_concat(input.reshape(b * l), ori_w, add_w, vocab=vocab)
  return out.reshape(b, l, d)
